# baseline (speedup 1.0000x reference)
import functools
import math

import jax
import jax.numpy as jnp
from jax import lax
from jax.experimental import pallas as pl
from jax.experimental.pallas import tpu as pltpu
from jax.experimental.pallas import tpu_sc as plsc

_K = 10
_NC, _NS, _L = 2, 16, 16
_NW = _NC * _NS
_NEG = -(2**31)
_BLK = 8
_GRP = 8
_P1U = 16


def _insert_one(ts, u):
    out = []
    for t in ts:
        hi = jnp.maximum(t, u)
        u = jnp.minimum(t, u)
        out.append(hi)
    return out


def _make_sc_loss(bq, n):
    qpw = bq // _NW
    nv = n // _L
    weights = [1.0 / math.log2(r + 2.0) for r in range(_K)]
    mesh = plsc.VectorSubcoreMesh(
        core_axis_name="c", subcore_axis_name="s",
        num_cores=_NC, num_subcores=_NS)

    @functools.partial(
        pl.kernel,
        out_type=jax.ShapeDtypeStruct((_NW, _L), jnp.float32),
        mesh=mesh,
        scratch_types=[
            pltpu.VMEM((1, n), jnp.float32),
            pltpu.VMEM((1, n), jnp.float32),
            pltpu.VMEM((1, n), jnp.float32),
            pltpu.VMEM((1, n), jnp.float32),
            pltpu.VMEM((1, _K * _L), jnp.int32),
            pltpu.VMEM((1, _L), jnp.float32),
            pltpu.SemaphoreType.DMA,
            pltpu.SemaphoreType.DMA,
            pltpu.SemaphoreType.DMA,
            pltpu.SemaphoreType.DMA,
        ],
        compiler_params=pltpu.CompilerParams(needs_layout_passes=False),
    )
    def sc_loss(scores_hbm, maskf_hbm, out_hbm, row_v0, row_v1, mw_v0, mw_v1,
                ts_v, acc_v, ss0, ss1, ms0, ms1):
        wid = lax.axis_index("s") * _NC + lax.axis_index("c")
        base = wid * qpw
        rows = (row_v0, row_v1)
        mws = (mw_v0, mw_v1)
        ssems = (ss0, ss1)
        msems = (ms0, ms1)
        lane = lax.iota(jnp.int32, _L)

        def start_row_dma(q, slot):
            pltpu.async_copy(scores_hbm.at[pl.ds(base + q, 1)], rows[slot],
                             ssems[slot])
            pltpu.async_copy(maskf_hbm.at[pl.ds(base + q, 1)], mws[slot],
                             msems[slot])

        def wait_row_dma(slot):
            pltpu.make_async_copy(scores_hbm.at[pl.ds(0, 1)], rows[slot],
                                  ssems[slot]).wait()
            pltpu.make_async_copy(maskf_hbm.at[pl.ds(0, 1)], mws[slot],
                                  msems[slot]).wait()

        start_row_dma(0, 0)

        def do_row(q, slot, loss_sum):
            row_r = rows[slot]
            mw_r = mws[slot]
            wait_row_dma(slot)

            @pl.when(q + 1 < qpw)
            def _prefetch():
                start_row_dma(q + 1, 1 - slot)

            def p1(i, accs):
                j0 = i * _P1U
                accs = list(accs)
                for u in range(_P1U):
                    v = row_r[0, pl.ds((j0 + u) * _L, _L)]
                    accs[u % 4] = jnp.maximum(accs[u % 4], v)
                return tuple(accs)

            ninf = jnp.full((_L,), -jnp.inf, jnp.float32)
            a0, a1, a2, a3 = lax.fori_loop(0, nv // _P1U, p1,
                                           (ninf, ninf, ninf, ninf))
            lm = jnp.maximum(jnp.maximum(a0, a1), jnp.maximum(a2, a3))

            sorted_lm = lax.sort(lm)
            tau = jnp.max(jnp.where(lane <= _L - _K, sorted_lm, -jnp.inf))

            negv = jnp.full((_L,), _NEG, jnp.int32)
            for t in range(_K):
                ts_v[0, pl.ds(t * _L, _L)] = negv

            def load_ts():
                return [ts_v[0, pl.ds(t * _L, _L)] for t in range(_K)]

            def store_ts(ts):
                for t in range(_K):
                    ts_v[0, pl.ds(t * _L, _L)] = ts[t]

            def insert_block(j0):
                vs = [row_r[0, pl.ds((j0 + u) * _L, _L)] for u in range(_BLK)]
                keeps = [v >= tau for v in vs]
                mkeys = []
                for u in range(_BLK):
                    b = plsc.bitcast(vs[u], jnp.int32)
                    key = jnp.where(b >= 0, b, b ^ jnp.int32(0x7FFFFFFF))
                    labbit = mw_r[0, pl.ds((j0 + u) * _L, _L)].astype(jnp.int32)
                    kv = (key & jnp.int32(-2)) | labbit
                    mkeys.append(jnp.where(keeps[u], kv, jnp.int32(_NEG)))
                cnt = keeps[0].astype(jnp.int32)
                for u in range(1, _BLK):
                    cnt = cnt + keeps[u].astype(jnp.int32)
                multi = jnp.any(cnt >= 2)

                @pl.when(jnp.logical_not(multi))
                def _fast():
                    k01 = jnp.maximum(mkeys[0], mkeys[1])
                    k23 = jnp.maximum(mkeys[2], mkeys[3])
                    k45 = jnp.maximum(mkeys[4], mkeys[5])
                    k67 = jnp.maximum(mkeys[6], mkeys[7])
                    merged = jnp.maximum(jnp.maximum(k01, k23),
                                         jnp.maximum(k45, k67))
                    store_ts(_insert_one(load_ts(), merged))

                @pl.when(multi)
                def _slow():
                    ts = load_ts()
                    for u in range(_BLK):
                        ts = _insert_one(ts, mkeys[u])
                    store_ts(ts)

            def p2(i, carry):
                g0 = i * (_GRP * _BLK)
                bms = []
                for blk in range(_GRP):
                    j0 = g0 + blk * _BLK
                    vs = [row_r[0, pl.ds((j0 + u) * _L, _L)]
                          for u in range(_BLK)]
                    m01 = jnp.maximum(vs[0], vs[1])
                    m23 = jnp.maximum(vs[2], vs[3])
                    m45 = jnp.maximum(vs[4], vs[5])
                    m67 = jnp.maximum(vs[6], vs[7])
                    bms.append(jnp.maximum(jnp.maximum(m01, m23),
                                           jnp.maximum(m45, m67)))
                gm = bms[0]
                for blk in range(1, _GRP):
                    gm = jnp.maximum(gm, bms[blk])
                ghit = jnp.any(gm >= tau)

                @pl.when(ghit)
                def _group():
                    for blk in range(_GRP):
                        hit = jnp.any(bms[blk] >= tau)

                        @pl.when(hit)
                        def _blk(j0=g0 + blk * _BLK):
                            insert_block(j0)

                return carry

            lax.fori_loop(0, nv // (_GRP * _BLK), p2, jnp.int32(0))
            ts = load_ts()

            actual = jnp.float32(0.0)
            mpos = jnp.int32(0)
            for r in range(_K):
                head = ts[0]
                m = jnp.max(head)
                lab = m & 1
                actual = actual + jnp.float32(weights[r]) * lab.astype(jnp.float32)
                mpos = mpos + lab
                sel = lane == plsc.all_reduce_ffs(head == m)
                for i in range(_K - 1):
                    ts[i] = jnp.where(sel, ts[i + 1], ts[i])
                ts[_K - 1] = jnp.where(sel, jnp.int32(_NEG), ts[_K - 1])

            recip = jnp.float32(0.0)
            for r in range(1, _K + 1):
                recip = jnp.where(mpos == r, jnp.float32(1.0 / sum(weights[:r])),
                                  recip)
            loss_q = jnp.where(mpos > 0, 1.0 - actual * recip, jnp.float32(0.0))
            return loss_sum + loss_q

        def pair_body(i, loss_sum):
            for b in range(2):
                loss_sum = do_row(i * 2 + b, b, loss_sum)
            return loss_sum

        loss_sum = lax.fori_loop(0, qpw // 2, pair_body, jnp.float32(0.0))
        acc_v[0, pl.ds(0, _L)] = jnp.broadcast_to(loss_sum, (_L,))
        pltpu.sync_copy(acc_v, out_hbm.at[pl.ds(wid, 1)])

    return sc_loss


@jax.jit
def _run_sc(scores, mask):
    bq, n = scores.shape
    maskf = mask.astype(jnp.float32)
    out = _make_sc_loss(bq, n)(scores, maskf)
    return jnp.sum(out) / jnp.float32(bq * _L)


def kernel(similarity_scores, positive_mask):
    return _run_sc(similarity_scores, positive_mask)


# speedup vs baseline: 1.3435x; 1.3435x over previous
import functools
import math

import jax
import jax.numpy as jnp
from jax import lax
from jax.experimental import pallas as pl
from jax.experimental.pallas import tpu as pltpu
from jax.experimental.pallas import tpu_sc as plsc

_K = 10
_NC, _NS, _L = 2, 16, 16
_NW = _NC * _NS
_NEG = -(2**31)
_BLK = 8
_GRP = 2
_P1U = 16


def _insert_one(ts, u):
    out = []
    for t in ts:
        hi = jnp.maximum(t, u)
        u = jnp.minimum(t, u)
        out.append(hi)
    return out


def _make_sc_loss(bq, n):
    qpw = bq // _NW
    nv = n // _L
    weights = [1.0 / math.log2(r + 2.0) for r in range(_K)]
    mesh = plsc.VectorSubcoreMesh(
        core_axis_name="c", subcore_axis_name="s",
        num_cores=_NC, num_subcores=_NS)

    @functools.partial(
        pl.kernel,
        out_type=jax.ShapeDtypeStruct((_NW, _L), jnp.float32),
        mesh=mesh,
        scratch_types=[
            pltpu.VMEM((1, n), jnp.float32),
            pltpu.VMEM((1, n), jnp.float32),
            pltpu.VMEM((1, n), jnp.float32),
            pltpu.VMEM((1, n), jnp.float32),
            pltpu.VMEM((1, _K * _L), jnp.int32),
            pltpu.VMEM((1, _L), jnp.float32),
            pltpu.SemaphoreType.DMA,
            pltpu.SemaphoreType.DMA,
            pltpu.SemaphoreType.DMA,
            pltpu.SemaphoreType.DMA,
        ],
        compiler_params=pltpu.CompilerParams(needs_layout_passes=False),
    )
    def sc_loss(scores_hbm, maskf_hbm, out_hbm, row_v0, row_v1, mw_v0, mw_v1,
                ts_v, acc_v, ss0, ss1, ms0, ms1):
        wid = lax.axis_index("s") * _NC + lax.axis_index("c")
        base = wid * qpw
        rows = (row_v0, row_v1)
        mws = (mw_v0, mw_v1)
        ssems = (ss0, ss1)
        msems = (ms0, ms1)
        lane = lax.iota(jnp.int32, _L)

        def start_row_dma(q, slot):
            pltpu.async_copy(scores_hbm.at[pl.ds(base + q, 1)], rows[slot],
                             ssems[slot])
            pltpu.async_copy(maskf_hbm.at[pl.ds(base + q, 1)], mws[slot],
                             msems[slot])

        def wait_row_dma(slot):
            pltpu.make_async_copy(scores_hbm.at[pl.ds(0, 1)], rows[slot],
                                  ssems[slot]).wait()
            pltpu.make_async_copy(maskf_hbm.at[pl.ds(0, 1)], mws[slot],
                                  msems[slot]).wait()

        start_row_dma(0, 0)

        def do_row(q, slot, loss_sum):
            row_r = rows[slot]
            mw_r = mws[slot]
            wait_row_dma(slot)

            @pl.when(q + 1 < qpw)
            def _prefetch():
                start_row_dma(q + 1, 1 - slot)

            def p1(i, accs):
                j0 = i * _P1U
                accs = list(accs)
                for u in range(_P1U):
                    v = row_r[0, pl.ds((j0 + u) * _L, _L)]
                    accs[u % 4] = jnp.maximum(accs[u % 4], v)
                return tuple(accs)

            ninf = jnp.full((_L,), -jnp.inf, jnp.float32)
            a0, a1, a2, a3 = lax.fori_loop(0, nv // _P1U, p1,
                                           (ninf, ninf, ninf, ninf))
            lm = jnp.maximum(jnp.maximum(a0, a1), jnp.maximum(a2, a3))

            sorted_lm = lax.sort(lm)
            tau = jnp.max(jnp.where(lane <= _L - _K, sorted_lm, -jnp.inf))

            negv = jnp.full((_L,), _NEG, jnp.int32)
            for t in range(_K):
                ts_v[0, pl.ds(t * _L, _L)] = negv

            def load_ts():
                return [ts_v[0, pl.ds(t * _L, _L)] for t in range(_K)]

            def store_ts(ts):
                for t in range(_K):
                    ts_v[0, pl.ds(t * _L, _L)] = ts[t]

            def insert_block(j0):
                vs = [row_r[0, pl.ds((j0 + u) * _L, _L)] for u in range(_BLK)]
                keeps = [v >= tau for v in vs]
                mkeys = []
                for u in range(_BLK):
                    b = plsc.bitcast(vs[u], jnp.int32)
                    key = jnp.where(b >= 0, b, b ^ jnp.int32(0x7FFFFFFF))
                    labbit = mw_r[0, pl.ds((j0 + u) * _L, _L)].astype(jnp.int32)
                    kv = (key & jnp.int32(-2)) | labbit
                    mkeys.append(jnp.where(keeps[u], kv, jnp.int32(_NEG)))
                cnt = keeps[0].astype(jnp.int32)
                for u in range(1, _BLK):
                    cnt = cnt + keeps[u].astype(jnp.int32)
                multi = jnp.any(cnt >= 2)

                @pl.when(jnp.logical_not(multi))
                def _fast():
                    k01 = jnp.maximum(mkeys[0], mkeys[1])
                    k23 = jnp.maximum(mkeys[2], mkeys[3])
                    k45 = jnp.maximum(mkeys[4], mkeys[5])
                    k67 = jnp.maximum(mkeys[6], mkeys[7])
                    merged = jnp.maximum(jnp.maximum(k01, k23),
                                         jnp.maximum(k45, k67))
                    store_ts(_insert_one(load_ts(), merged))

                @pl.when(multi)
                def _slow():
                    ts = load_ts()
                    for u in range(_BLK):
                        ts = _insert_one(ts, mkeys[u])
                    store_ts(ts)

            def p2(i, carry):
                g0 = i * (_GRP * _BLK)
                bms = []
                for blk in range(_GRP):
                    j0 = g0 + blk * _BLK
                    vs = [row_r[0, pl.ds((j0 + u) * _L, _L)]
                          for u in range(_BLK)]
                    m01 = jnp.maximum(vs[0], vs[1])
                    m23 = jnp.maximum(vs[2], vs[3])
                    m45 = jnp.maximum(vs[4], vs[5])
                    m67 = jnp.maximum(vs[6], vs[7])
                    bms.append(jnp.maximum(jnp.maximum(m01, m23),
                                           jnp.maximum(m45, m67)))
                gm = bms[0]
                for blk in range(1, _GRP):
                    gm = jnp.maximum(gm, bms[blk])
                ghit = jnp.any(gm >= tau)

                @pl.when(ghit)
                def _group():
                    for blk in range(_GRP):
                        hit = jnp.any(bms[blk] >= tau)

                        @pl.when(hit)
                        def _blk(j0=g0 + blk * _BLK):
                            insert_block(j0)

                return carry

            lax.fori_loop(0, nv // (_GRP * _BLK), p2, jnp.int32(0))
            ts = load_ts()

            actual = jnp.float32(0.0)
            mpos = jnp.int32(0)
            for r in range(_K):
                head = ts[0]
                m = jnp.max(head)
                lab = m & 1
                actual = actual + jnp.float32(weights[r]) * lab.astype(jnp.float32)
                mpos = mpos + lab
                sel = lane == plsc.all_reduce_ffs(head == m)
                for i in range(_K - 1):
                    ts[i] = jnp.where(sel, ts[i + 1], ts[i])
                ts[_K - 1] = jnp.where(sel, jnp.int32(_NEG), ts[_K - 1])

            recip = jnp.float32(0.0)
            for r in range(1, _K + 1):
                recip = jnp.where(mpos == r, jnp.float32(1.0 / sum(weights[:r])),
                                  recip)
            loss_q = jnp.where(mpos > 0, 1.0 - actual * recip, jnp.float32(0.0))
            return loss_sum + loss_q

        def pair_body(i, loss_sum):
            for b in range(2):
                loss_sum = do_row(i * 2 + b, b, loss_sum)
            return loss_sum

        loss_sum = lax.fori_loop(0, qpw // 2, pair_body, jnp.float32(0.0))
        acc_v[0, pl.ds(0, _L)] = jnp.broadcast_to(loss_sum, (_L,))
        pltpu.sync_copy(acc_v, out_hbm.at[pl.ds(wid, 1)])

    return sc_loss


@jax.jit
def _run_sc(scores, mask):
    bq, n = scores.shape
    maskf = mask.astype(jnp.float32)
    out = _make_sc_loss(bq, n)(scores, maskf)
    return jnp.sum(out) / jnp.float32(bq * _L)


def kernel(similarity_scores, positive_mask):
    return _run_sc(similarity_scores, positive_mask)


# speedup vs baseline: 1.3862x; 1.0317x over previous
import functools
import math

import jax
import jax.numpy as jnp
from jax import lax
from jax.experimental import pallas as pl
from jax.experimental.pallas import tpu as pltpu
from jax.experimental.pallas import tpu_sc as plsc

_K = 10
_NC, _NS, _L = 2, 16, 16
_NW = _NC * _NS
_NEG = -(2**31)
_BLK = 8
_GRP = 4
_P1U = 16


def _insert_one(ts, u):
    out = []
    for t in ts:
        hi = jnp.maximum(t, u)
        u = jnp.minimum(t, u)
        out.append(hi)
    return out


def _make_sc_loss(bq, n):
    qpw = bq // _NW
    nv = n // _L
    weights = [1.0 / math.log2(r + 2.0) for r in range(_K)]
    mesh = plsc.VectorSubcoreMesh(
        core_axis_name="c", subcore_axis_name="s",
        num_cores=_NC, num_subcores=_NS)

    @functools.partial(
        pl.kernel,
        out_type=jax.ShapeDtypeStruct((_NW, _L), jnp.float32),
        mesh=mesh,
        scratch_types=[
            pltpu.VMEM((1, n), jnp.float32),
            pltpu.VMEM((1, n), jnp.float32),
            pltpu.VMEM((1, n), jnp.float32),
            pltpu.VMEM((1, n), jnp.float32),
            pltpu.VMEM((1, _K * _L), jnp.int32),
            pltpu.VMEM((1, _L), jnp.float32),
            pltpu.SemaphoreType.DMA,
            pltpu.SemaphoreType.DMA,
            pltpu.SemaphoreType.DMA,
            pltpu.SemaphoreType.DMA,
        ],
        compiler_params=pltpu.CompilerParams(needs_layout_passes=False),
    )
    def sc_loss(scores_hbm, maskf_hbm, out_hbm, row_v0, row_v1, mw_v0, mw_v1,
                ts_v, acc_v, ss0, ss1, ms0, ms1):
        wid = lax.axis_index("s") * _NC + lax.axis_index("c")
        base = wid * qpw
        rows = (row_v0, row_v1)
        mws = (mw_v0, mw_v1)
        ssems = (ss0, ss1)
        msems = (ms0, ms1)
        lane = lax.iota(jnp.int32, _L)

        def start_row_dma(q, slot):
            pltpu.async_copy(scores_hbm.at[pl.ds(base + q, 1)], rows[slot],
                             ssems[slot])
            pltpu.async_copy(maskf_hbm.at[pl.ds(base + q, 1)], mws[slot],
                             msems[slot])

        def wait_row_dma(slot):
            pltpu.make_async_copy(scores_hbm.at[pl.ds(0, 1)], rows[slot],
                                  ssems[slot]).wait()
            pltpu.make_async_copy(maskf_hbm.at[pl.ds(0, 1)], mws[slot],
                                  msems[slot]).wait()

        start_row_dma(0, 0)

        def do_row(q, slot, loss_sum):
            row_r = rows[slot]
            mw_r = mws[slot]
            wait_row_dma(slot)

            @pl.when(q + 1 < qpw)
            def _prefetch():
                start_row_dma(q + 1, 1 - slot)

            def p1(i, accs):
                j0 = i * _P1U
                accs = list(accs)
                for u in range(_P1U):
                    v = row_r[0, pl.ds((j0 + u) * _L, _L)]
                    accs[u % 4] = jnp.maximum(accs[u % 4], v)
                return tuple(accs)

            ninf = jnp.full((_L,), -jnp.inf, jnp.float32)
            a0, a1, a2, a3 = lax.fori_loop(0, nv // _P1U, p1,
                                           (ninf, ninf, ninf, ninf))
            lm = jnp.maximum(jnp.maximum(a0, a1), jnp.maximum(a2, a3))

            sorted_lm = lax.sort(lm)
            tau = jnp.max(jnp.where(lane <= _L - _K, sorted_lm, -jnp.inf))

            negv = jnp.full((_L,), _NEG, jnp.int32)
            for t in range(_K):
                ts_v[0, pl.ds(t * _L, _L)] = negv

            def load_ts():
                return [ts_v[0, pl.ds(t * _L, _L)] for t in range(_K)]

            def store_ts(ts):
                for t in range(_K):
                    ts_v[0, pl.ds(t * _L, _L)] = ts[t]

            def insert_block(j0):
                vs = [row_r[0, pl.ds((j0 + u) * _L, _L)] for u in range(_BLK)]
                keeps = [v >= tau for v in vs]
                mkeys = []
                for u in range(_BLK):
                    b = plsc.bitcast(vs[u], jnp.int32)
                    key = jnp.where(b >= 0, b, b ^ jnp.int32(0x7FFFFFFF))
                    labbit = mw_r[0, pl.ds((j0 + u) * _L, _L)].astype(jnp.int32)
                    kv = (key & jnp.int32(-2)) | labbit
                    mkeys.append(jnp.where(keeps[u], kv, jnp.int32(_NEG)))
                cnt = keeps[0].astype(jnp.int32)
                for u in range(1, _BLK):
                    cnt = cnt + keeps[u].astype(jnp.int32)
                multi = jnp.any(cnt >= 2)

                @pl.when(jnp.logical_not(multi))
                def _fast():
                    k01 = jnp.maximum(mkeys[0], mkeys[1])
                    k23 = jnp.maximum(mkeys[2], mkeys[3])
                    k45 = jnp.maximum(mkeys[4], mkeys[5])
                    k67 = jnp.maximum(mkeys[6], mkeys[7])
                    merged = jnp.maximum(jnp.maximum(k01, k23),
                                         jnp.maximum(k45, k67))
                    store_ts(_insert_one(load_ts(), merged))

                @pl.when(multi)
                def _slow():
                    ts = load_ts()
                    for u in range(_BLK):
                        ts = _insert_one(ts, mkeys[u])
                    store_ts(ts)

            def p2(i, carry):
                g0 = i * (_GRP * _BLK)
                bms = []
                for blk in range(_GRP):
                    j0 = g0 + blk * _BLK
                    vs = [row_r[0, pl.ds((j0 + u) * _L, _L)]
                          for u in range(_BLK)]
                    m01 = jnp.maximum(vs[0], vs[1])
                    m23 = jnp.maximum(vs[2], vs[3])
                    m45 = jnp.maximum(vs[4], vs[5])
                    m67 = jnp.maximum(vs[6], vs[7])
                    bms.append(jnp.maximum(jnp.maximum(m01, m23),
                                           jnp.maximum(m45, m67)))
                gm = bms[0]
                for blk in range(1, _GRP):
                    gm = jnp.maximum(gm, bms[blk])
                ghit = jnp.any(gm >= tau)

                @pl.when(ghit)
                def _group():
                    for blk in range(_GRP):
                        hit = jnp.any(bms[blk] >= tau)

                        @pl.when(hit)
                        def _blk(j0=g0 + blk * _BLK):
                            insert_block(j0)

                return carry

            lax.fori_loop(0, nv // (_GRP * _BLK), p2, jnp.int32(0))
            ts = load_ts()

            actual = jnp.float32(0.0)
            mpos = jnp.int32(0)
            for r in range(_K):
                head = ts[0]
                m = jnp.max(head)
                lab = m & 1
                actual = actual + jnp.float32(weights[r]) * lab.astype(jnp.float32)
                mpos = mpos + lab
                sel = lane == plsc.all_reduce_ffs(head == m)
                for i in range(_K - 1):
                    ts[i] = jnp.where(sel, ts[i + 1], ts[i])
                ts[_K - 1] = jnp.where(sel, jnp.int32(_NEG), ts[_K - 1])

            recip = jnp.float32(0.0)
            for r in range(1, _K + 1):
                recip = jnp.where(mpos == r, jnp.float32(1.0 / sum(weights[:r])),
                                  recip)
            loss_q = jnp.where(mpos > 0, 1.0 - actual * recip, jnp.float32(0.0))
            return loss_sum + loss_q

        def pair_body(i, loss_sum):
            for b in range(2):
                loss_sum = do_row(i * 2 + b, b, loss_sum)
            return loss_sum

        loss_sum = lax.fori_loop(0, qpw // 2, pair_body, jnp.float32(0.0))
        acc_v[0, pl.ds(0, _L)] = jnp.broadcast_to(loss_sum, (_L,))
        pltpu.sync_copy(acc_v, out_hbm.at[pl.ds(wid, 1)])

    return sc_loss


@jax.jit
def _run_sc(scores, mask):
    bq, n = scores.shape
    maskf = mask.astype(jnp.float32)
    out = _make_sc_loss(bq, n)(scores, maskf)
    return jnp.sum(out) / jnp.float32(bq * _L)


def kernel(similarity_scores, positive_mask):
    return _run_sc(similarity_scores, positive_mask)
